# baseline (speedup 1.0000x reference)
import functools

import jax
import jax.numpy as jnp
from jax.experimental import pallas as pl
from jax.experimental.pallas import tpu as pltpu

_EPS = 1e-12
_GAP = 64


def _fused_body(w_ref, u_ref, b_ref, x_ref, o_ref, u_out_ref,
                pad_ref, inv_sig_ref, *, height, width):
    imgs = x_ref.shape[0]
    cin = pad_ref.shape[0]
    hw = height * width
    slot = hw + _GAP

    @pl.when(pl.program_id(0) == 0)
    def _prologue():
        w = w_ref[...]
        u = u_ref[...].astype(jnp.bfloat16)
        v = jnp.dot(u, w, preferred_element_type=jnp.float32)
        v = v * (1.0 / (jnp.sqrt(jnp.sum(v * v, keepdims=True)) + _EPS))
        wv = jax.lax.dot_general(v.astype(jnp.bfloat16), w,
                                 (((1,), (1,)), ((), ())),
                                 preferred_element_type=jnp.float32)
        u_new = wv * (1.0 / (jnp.sqrt(jnp.sum(wv * wv, keepdims=True)) + _EPS))
        sigma = jnp.sum(u_new * wv)
        inv_sig_ref[0, 0] = 1.0 / sigma
        u_out_ref[...] = u_new
        pad_ref[:, 0:_GAP] = jnp.zeros((cin, _GAP), jnp.bfloat16)
        for i in range(imgs):
            pad_ref[:, _GAP + i * slot + hw: _GAP + (i + 1) * slot] = (
                jnp.zeros((cin, _GAP), jnp.bfloat16))

    for i in range(imgs):
        pad_ref[:, _GAP + i * slot: _GAP + i * slot + hw] = (
            x_ref[i].astype(jnp.bfloat16))

    ww = jax.lax.broadcasted_iota(jnp.int32, (1, hw), 1) % width
    left_ok = ww >= 1
    right_ok = ww <= width - 2

    inv_sigma = inv_sig_ref[0, 0]
    for i in range(imgs):
        base = _GAP + i * slot
        acc = None
        for kh in range(3):
            for kw in range(3):
                d = (kh - 1) * width + (kw - 1)
                t = kh * 3 + kw
                xs = pad_ref[:, base + d: base + d + hw]
                if kw == 0:
                    xs = jnp.where(left_ok, xs, jnp.bfloat16(0))
                elif kw == 2:
                    xs = jnp.where(right_ok, xs, jnp.bfloat16(0))
                part = jnp.dot(w_ref[:, t * cin: (t + 1) * cin], xs,
                               preferred_element_type=jnp.float32)
                acc = part if acc is None else acc + part
        o_ref[i] = acc * inv_sigma + b_ref[...]


@jax.jit
def _forward(x, w_bar, bias, u):
    n, c, h, w = x.shape
    cout = w_bar.shape[0]
    k = c * w_bar.shape[2] * w_bar.shape[3]
    hw = h * w
    imgs_per_step = 2 if n % 2 == 0 else 1

    w_tap = w_bar.transpose(0, 2, 3, 1).reshape(cout, k).astype(jnp.bfloat16)

    body = functools.partial(_fused_body, height=h, width=w)
    out, u_new = pl.pallas_call(
        body,
        out_shape=(
            jax.ShapeDtypeStruct((n, cout, hw), jnp.float32),
            jax.ShapeDtypeStruct((1, cout), jnp.float32),
        ),
        grid=(n // imgs_per_step,),
        in_specs=[
            pl.BlockSpec((cout, k), lambda i: (0, 0)),
            pl.BlockSpec((1, cout), lambda i: (0, 0)),
            pl.BlockSpec((cout, 1), lambda i: (0, 0)),
            pl.BlockSpec((imgs_per_step, c, hw), lambda i: (i, 0, 0)),
        ],
        out_specs=(
            pl.BlockSpec((imgs_per_step, cout, hw), lambda i: (i, 0, 0)),
            pl.BlockSpec((1, cout), lambda i: (0, 0)),
        ),
        scratch_shapes=[
            pltpu.VMEM((c, _GAP + imgs_per_step * (hw + _GAP)), jnp.bfloat16),
            pltpu.SMEM((1, 1), jnp.float32),
        ],
        compiler_params=pltpu.CompilerParams(
            dimension_semantics=("arbitrary",),
        ),
    )(w_tap, u.reshape(1, cout), bias.reshape(cout, 1), x.reshape(n, c, hw))
    return out.reshape(n, cout, h, w), u_new.reshape(cout)


def kernel(x, w_bar, bias, u):
    return _forward(x, w_bar, bias, u)


# speedup vs baseline: 1.0016x; 1.0016x over previous
import functools

import jax
import jax.numpy as jnp
from jax.experimental import pallas as pl
from jax.experimental.pallas import tpu as pltpu

_EPS = 1e-12
_GAP = 64


def _fused_body(w_ref, u_ref, b_ref, x_ref, o_ref, u_out_ref,
                pad_ref, inv_sig_ref, *, height, width):
    imgs = x_ref.shape[0]
    cin = pad_ref.shape[0]
    hw = height * width
    slot = hw + _GAP

    @pl.when(pl.program_id(0) == 0)
    def _prologue():
        w = w_ref[...]
        u = u_ref[...].astype(jnp.bfloat16)
        v = jnp.dot(u, w, preferred_element_type=jnp.float32)
        v = v * (1.0 / (jnp.sqrt(jnp.sum(v * v, keepdims=True)) + _EPS))
        wv = jax.lax.dot_general(v.astype(jnp.bfloat16), w,
                                 (((1,), (1,)), ((), ())),
                                 preferred_element_type=jnp.float32)
        u_new = wv * (1.0 / (jnp.sqrt(jnp.sum(wv * wv, keepdims=True)) + _EPS))
        sigma = jnp.sum(u_new * wv)
        inv_sig_ref[0, 0] = 1.0 / sigma
        u_out_ref[...] = u_new
        pad_ref[:, 0:_GAP] = jnp.zeros((cin, _GAP), jnp.bfloat16)
        for i in range(imgs):
            pad_ref[:, _GAP + i * slot + hw: _GAP + (i + 1) * slot] = (
                jnp.zeros((cin, _GAP), jnp.bfloat16))

    for i in range(imgs):
        pad_ref[:, _GAP + i * slot: _GAP + i * slot + hw] = (
            x_ref[i].astype(jnp.bfloat16))

    ww = jax.lax.broadcasted_iota(jnp.int32, (1, hw), 1) % width
    left_ok = (ww >= 1).astype(jnp.bfloat16)
    right_ok = (ww <= width - 2).astype(jnp.bfloat16)

    inv_sigma = inv_sig_ref[0, 0]
    for i in range(imgs):
        base = _GAP + i * slot
        acc = None
        for kh in range(3):
            for kw in range(3):
                d = (kh - 1) * width + (kw - 1)
                t = kh * 3 + kw
                xs = pad_ref[:, base + d: base + d + hw]
                if kw == 0:
                    xs = xs * left_ok
                elif kw == 2:
                    xs = xs * right_ok
                part = jnp.dot(w_ref[:, t * cin: (t + 1) * cin], xs,
                               preferred_element_type=jnp.float32)
                acc = part if acc is None else acc + part
        o_ref[i] = acc * inv_sigma + b_ref[...]


@jax.jit
def _forward(x, w_bar, bias, u):
    n, c, h, w = x.shape
    cout = w_bar.shape[0]
    k = c * w_bar.shape[2] * w_bar.shape[3]
    hw = h * w
    imgs_per_step = 2 if n % 2 == 0 else 1

    w_tap = w_bar.transpose(0, 2, 3, 1).reshape(cout, k).astype(jnp.bfloat16)

    body = functools.partial(_fused_body, height=h, width=w)
    out, u_new = pl.pallas_call(
        body,
        out_shape=(
            jax.ShapeDtypeStruct((n, cout, hw), jnp.float32),
            jax.ShapeDtypeStruct((1, cout), jnp.float32),
        ),
        grid=(n // imgs_per_step,),
        in_specs=[
            pl.BlockSpec((cout, k), lambda i: (0, 0)),
            pl.BlockSpec((1, cout), lambda i: (0, 0)),
            pl.BlockSpec((cout, 1), lambda i: (0, 0)),
            pl.BlockSpec((imgs_per_step, c, hw), lambda i: (i, 0, 0)),
        ],
        out_specs=(
            pl.BlockSpec((imgs_per_step, cout, hw), lambda i: (i, 0, 0)),
            pl.BlockSpec((1, cout), lambda i: (0, 0)),
        ),
        scratch_shapes=[
            pltpu.VMEM((c, _GAP + imgs_per_step * (hw + _GAP)), jnp.bfloat16),
            pltpu.SMEM((1, 1), jnp.float32),
        ],
        compiler_params=pltpu.CompilerParams(
            dimension_semantics=("arbitrary",),
        ),
    )(w_tap, u.reshape(1, cout), bias.reshape(cout, 1), x.reshape(n, c, hw))
    return out.reshape(n, cout, h, w), u_new.reshape(cout)


def kernel(x, w_bar, bias, u):
    return _forward(x, w_bar, bias, u)
